# Initial kernel scaffold; baseline (speedup 1.0000x reference)
import jax
import jax.numpy as jnp
from jax.experimental import pallas as pl


def kernel(points, lattice_type, table):
    raise NotImplementedError("write your pallas kernel here")


# speedup vs baseline: 16.5588x; 16.5588x over previous
import functools
import math

import jax
import jax.numpy as jnp
import numpy as np
from jax import lax
from jax.experimental import pallas as pl
from jax.experimental.pallas import tpu as pltpu
from jax.experimental.pallas import tpu_sc as plsc

N_LEVELS = 16
N_FEATS = 2
LOG2_T = 19
T = 2 ** LOG2_T
BASE_RES = 4
PER_LEVEL_SCALE = 1.5
P2 = 2654435761
P3 = 805459861

NC = 2
NS = 16
NW = NC * NS

N_PTS = 524288
W = N_PTS // NW
C = 512
C2 = 2 * C
NCH = W // C
NG = C2 // 16
SB = C2 // 128
GPR = 128 // 16

_RES = [int(math.floor(BASE_RES * (PER_LEVEL_SCALE ** l))) for l in range(N_LEVELS)]
_DENSE = [(r + 1) ** 3 <= T for r in _RES]
K2 = np.uint32(P2).view(np.int32).item()
K3 = np.uint32(P3).view(np.int32).item()
MASK = T - 1


def _body(xd, yd, zd, tflat, out, xdv, ydv, zdv, w2, outv, sem, idx2, rows2):
    wid = lax.axis_index("s") * NC + lax.axis_index("c")
    i16 = lax.iota(jnp.int32, 16)
    parv = i16 & 1

    def chunk_body(ch, _):
        src0 = pl.multiple_of(wid * (2 * W) + ch * C2, 8)
        pltpu.sync_copy(xd.at[pl.ds(src0, C2)], xdv)
        pltpu.sync_copy(yd.at[pl.ds(src0, C2)], ydv)
        pltpu.sync_copy(zd.at[pl.ds(src0, C2)], zdv)

        for l in range(N_LEVELS):
            res = _RES[l]
            dense = _DENSE[l]

            def idx_body(g, _, l=l, res=res, dense=dense):
                gs = pl.ds(g * 16, 16)
                xv = xdv[gs]
                yv = ydv[gs]
                zv = zdv[gs]
                rf = jnp.float32(res)
                px = xv * rf
                py = yv * rf
                pz = zv * rf
                cx = px.astype(jnp.int32)
                cy = py.astype(jnp.int32)
                cz = pz.astype(jnp.int32)
                wx = px - cx.astype(jnp.float32)
                wy = py - cy.astype(jnp.float32)
                wz = pz - cz.astype(jnp.float32)
                rm1 = jnp.int32(res - 1)
                z32 = jnp.int32(0)
                cx = jnp.minimum(jnp.maximum(cx, z32), rm1)
                cy = jnp.minimum(jnp.maximum(cy, z32), rm1)
                cz = jnp.minimum(jnp.maximum(cz, z32), rm1)
                ux = 1.0 - wx
                uy = 1.0 - wy
                uz = 1.0 - wz
                sxs = (ux, wx)
                pyz = [[None, None], [None, None]]
                for by in range(2):
                    for bz in range(2):
                        sy = wy if by else uy
                        sz = wz if bz else uz
                        pyz[by][bz] = sy * sz
                base = parv + jnp.int32(2 * l * T)
                row0 = g // GPR
                col = pl.ds((g % GPR) * 16, 16)
                if dense:
                    m1 = res + 1
                    m2 = m1 * m1
                    ax = (cx * 2 + base, cx * 2 + (base + 2))
                    by_ = (cy * jnp.int32(2 * m1), (cy + 1) * jnp.int32(2 * m1))
                    bz_ = (cz * jnp.int32(2 * m2), (cz + 1) * jnp.int32(2 * m2))
                    for n in range(8):
                        bx, by, bz = (n >> 2) & 1, (n >> 1) & 1, n & 1
                        iv = ax[bx] + by_[by] + bz_[bz]
                        wv = sxs[bx] * pyz[by][bz]
                        idx2[row0 + n * SB, col] = iv
                        w2[n, gs] = wv
                else:
                    hx = (cx, cx + 1)
                    hy0 = cy * jnp.int32(K2)
                    hz0 = cz * jnp.int32(K3)
                    hy = (hy0, hy0 + jnp.int32(K2))
                    hz = (hz0, hz0 + jnp.int32(K3))
                    eyz = [[hy[a] ^ hz[b] for b in range(2)] for a in range(2)]
                    for n in range(8):
                        bx, by, bz = (n >> 2) & 1, (n >> 1) & 1, n & 1
                        t = (hx[bx] ^ eyz[by][bz]) & jnp.int32(MASK)
                        iv = t * 2 + base
                        wv = sxs[bx] * pyz[by][bz]
                        idx2[row0 + n * SB, col] = iv
                        w2[n, gs] = wv
                return _

            lax.fori_loop(0, NG, idx_body, None)

            def fire_body(r, _):
                pltpu.async_copy(tflat.at[idx2.at[r]], rows2.at[r], sem)
                return _

            lax.fori_loop(0, 8 * SB, fire_body, None)

            def drain_body(r, _):
                pltpu.make_async_copy(tflat.at[idx2.at[r]], rows2.at[r], sem).wait()
                return _

            lax.fori_loop(0, 8 * SB, drain_body, None)

            def acc_body(g, _, l=l):
                gs = pl.ds(g * 16, 16)
                row0 = g // GPR
                col = pl.ds((g % GPR) * 16, 16)
                a = rows2[row0, col] * w2[0, gs]
                for n in range(1, 8):
                    a = a + rows2[row0 + n * SB, col] * w2[n, gs]
                outv[l, gs] = a
                return _

            lax.fori_loop(0, NG, acc_body, None)

        dst0 = pl.multiple_of((wid * NCH + ch) * N_LEVELS, 8)
        pltpu.sync_copy(outv, out.at[pl.ds(dst0, N_LEVELS), :])
        return _

    lax.fori_loop(0, NCH, chunk_body, None)


@functools.partial(
    pl.kernel,
    mesh=plsc.VectorSubcoreMesh(core_axis_name="c", subcore_axis_name="s"),
    out_type=jax.ShapeDtypeStruct((NW * NCH * N_LEVELS, C2), jnp.float32),
    scratch_types=[
        pltpu.VMEM((C2,), jnp.float32),
        pltpu.VMEM((C2,), jnp.float32),
        pltpu.VMEM((C2,), jnp.float32),
        pltpu.VMEM((8, C2), jnp.float32),
        pltpu.VMEM((N_LEVELS, C2), jnp.float32),
        pltpu.SemaphoreType.DMA,
        pltpu.VMEM((8 * SB, 128), jnp.int32),
        pltpu.VMEM((8 * SB, 128), jnp.float32),
    ],
)
def _encode(xd, yd, zd, tflat, out, *scratch):
    _body(xd, yd, zd, tflat, out, *scratch)


def kernel(points, lattice_type, table):
    del lattice_type
    dup = jnp.repeat(points, 2, axis=0).reshape(points.shape[0], 2, 3)
    xd = dup[:, :, 0].reshape(-1)
    yd = dup[:, :, 1].reshape(-1)
    zd = dup[:, :, 2].reshape(-1)
    tflat = table.reshape(-1)
    out = _encode(xd, yd, zd, tflat)
    out = out.reshape(NW * NCH, N_LEVELS, C, 2)
    out = out.transpose(0, 2, 1, 3)
    return out.reshape(N_PTS, N_LEVELS * N_FEATS)


# speedup vs baseline: 38.5628x; 2.3288x over previous
import functools
import math

import jax
import jax.numpy as jnp
import numpy as np
from jax import lax
from jax.experimental import pallas as pl
from jax.experimental.pallas import tpu as pltpu
from jax.experimental.pallas import tpu_sc as plsc

N_LEVELS = 16
N_FEATS = 2
LOG2_T = 19
T = 2 ** LOG2_T
BASE_RES = 4
PER_LEVEL_SCALE = 1.5
P2 = 2654435761
P3 = 805459861

NC = 2
NS = 16
NW = NC * NS

N_PTS = 524288
W = N_PTS // NW
C = 512
C2 = 2 * C
NCH = W // C
NG = C2 // 16
SB = C2 // 128
GPR = 128 // 16

_RES = [int(math.floor(BASE_RES * (PER_LEVEL_SCALE ** l))) for l in range(N_LEVELS)]
K2 = np.uint32(P2).view(np.int32).item()
K3 = np.uint32(P3).view(np.int32).item()
MASK = T - 1


def _pad16(n):
    return (n + 15) & ~15


LOC_LEVELS = (0, 1, 2, 3, 4)
SPM_LEVELS = (5, 6)
HBM_LEVELS = tuple(range(7, 16))
STREAM_LEVELS = SPM_LEVELS + HBM_LEVELS

_loc_off = {}
_loc_sz = {}
_o = 0
for _l in LOC_LEVELS:
    _loc_off[_l] = _o
    _loc_sz[_l] = _pad16(2 * (_RES[_l] + 1) ** 3)
    _o += _loc_sz[_l]
LOC_WORDS = _o

BS = 8192
_sp_off = {}
_sp_sz = {}
_o = 0
for _l in SPM_LEVELS:
    _sp_off[_l] = _o
    _sp_sz[_l] = (2 * (_RES[_l] + 1) ** 3 + BS - 1) // BS * BS
    _o += _sp_sz[_l]
SPM_WORDS = _o


def _body(xd, yd, zd, tflat, out, xdv, ydv, zdv, tabv, spm, bounce,
          w2a, w2b, outv, sema, semb, idx2a, idx2b, rows2a, rows2b):
    w2s = (w2a, w2b)
    idx2s = (idx2a, idx2b)
    rows2s = (rows2a, rows2b)
    sems = (sema, semb)
    sid = lax.axis_index("s")
    wid = sid * NC + lax.axis_index("c")
    i16 = lax.iota(jnp.int32, 16)
    parv = i16 & 1

    for l in LOC_LEVELS:
        pltpu.sync_copy(tflat.at[pl.ds(2 * l * T, _loc_sz[l])],
                        tabv.at[pl.ds(_loc_off[l], _loc_sz[l])])

    for l in SPM_LEVELS:
        nb = _sp_sz[l] // BS

        def stage_body(k, _, l=l):
            @pl.when(sid == k % NS)
            def _do():
                src = pl.multiple_of(2 * l * T + k * BS, 8)
                dst = pl.multiple_of(_sp_off[l] + k * BS, 8)
                pltpu.sync_copy(tflat.at[pl.ds(src, BS)], bounce)
                pltpu.sync_copy(bounce, spm.at[pl.ds(dst, BS)])
            return _

        lax.fori_loop(0, nb, stage_body, None)

    plsc.subcore_barrier()

    def corner_math(g, l):
        res = _RES[l]
        gs = pl.ds(g * 16, 16)
        xv = xdv[gs]
        yv = ydv[gs]
        zv = zdv[gs]
        rf = jnp.float32(res)
        px = xv * rf
        py = yv * rf
        pz = zv * rf
        cx = px.astype(jnp.int32)
        cy = py.astype(jnp.int32)
        cz = pz.astype(jnp.int32)
        wx = px - cx.astype(jnp.float32)
        wy = py - cy.astype(jnp.float32)
        wz = pz - cz.astype(jnp.float32)
        rm1 = jnp.int32(res - 1)
        z32 = jnp.int32(0)
        cx = jnp.minimum(jnp.maximum(cx, z32), rm1)
        cy = jnp.minimum(jnp.maximum(cy, z32), rm1)
        cz = jnp.minimum(jnp.maximum(cz, z32), rm1)
        ux = 1.0 - wx
        uy = 1.0 - wy
        uz = 1.0 - wz
        sxs = (ux, wx)
        pyz = [[(wy if by else uy) * (wz if bz else uz) for bz in range(2)]
               for by in range(2)]

        dense = (res + 1) ** 3 <= T
        if dense:
            if l in _loc_off:
                base = parv + jnp.int32(_loc_off[l])
            elif l in _sp_off:
                base = parv + jnp.int32(_sp_off[l])
            else:
                base = parv + jnp.int32(2 * l * T)
            m1 = res + 1
            m2 = m1 * m1
            ax = (cx * 2 + base, cx * 2 + (base + 2))
            by_ = (cy * jnp.int32(2 * m1), (cy + 1) * jnp.int32(2 * m1))
            bz_ = (cz * jnp.int32(2 * m2), (cz + 1) * jnp.int32(2 * m2))

            def iv(n):
                bx, by, bz = (n >> 2) & 1, (n >> 1) & 1, n & 1
                return ax[bx] + by_[by] + bz_[bz]
        else:
            base = parv + jnp.int32(2 * l * T)
            hx = (cx, cx + 1)
            hy0 = cy * jnp.int32(K2)
            hz0 = cz * jnp.int32(K3)
            hy = (hy0, hy0 + jnp.int32(K2))
            hz = (hz0, hz0 + jnp.int32(K3))
            eyz = [[hy[a] ^ hz[b] for b in range(2)] for a in range(2)]

            def iv(n):
                bx, by, bz = (n >> 2) & 1, (n >> 1) & 1, n & 1
                t = (hx[bx] ^ eyz[by][bz]) & jnp.int32(MASK)
                return t * 2 + base

        def wv(n):
            bx, by, bz = (n >> 2) & 1, (n >> 1) & 1, n & 1
            return sxs[bx] * pyz[by][bz]

        return gs, iv, wv

    def chunk_body(ch, _):
        src0 = pl.multiple_of(wid * (2 * W) + ch * C2, 8)
        pltpu.sync_copy(xd.at[pl.ds(src0, C2)], xdv)
        pltpu.sync_copy(yd.at[pl.ds(src0, C2)], ydv)
        pltpu.sync_copy(zd.at[pl.ds(src0, C2)], zdv)

        def compute(l, s):
            idx2 = idx2s[s]
            w2 = w2s[s]

            def idx_body(g, _, l=l, idx2=idx2, w2=w2):
                gs, iv, wv = corner_math(g, l)
                row0 = g // GPR
                col = pl.ds((g % GPR) * 16, 16)
                for n in range(8):
                    idx2[row0 + n * SB, col] = iv(n)
                    w2[n, gs] = wv(n)
                return _

            lax.fori_loop(0, NG, idx_body, None)

        def fire(l, s):
            src = spm if l in _sp_off else tflat
            idx2 = idx2s[s]
            rows2 = rows2s[s]
            sem = sems[s]

            def fire_body(r, _, src=src, idx2=idx2, rows2=rows2, sem=sem):
                pltpu.async_copy(src.at[idx2.at[r]], rows2.at[r], sem)
                return _

            lax.fori_loop(0, 8 * SB, fire_body, None)

        def drain(s):
            idx2 = idx2s[s]
            rows2 = rows2s[s]
            sem = sems[s]

            def drain_body(r, _, idx2=idx2, rows2=rows2, sem=sem):
                pltpu.make_async_copy(tflat.at[idx2.at[r]], rows2.at[r],
                                      sem).wait()
                return _

            lax.fori_loop(0, 8 * SB, drain_body, None)

        def acc(l, s):
            rows2 = rows2s[s]
            w2 = w2s[s]

            def acc_body(g, _, l=l, rows2=rows2, w2=w2):
                gs = pl.ds(g * 16, 16)
                row0 = g // GPR
                col = pl.ds((g % GPR) * 16, 16)
                a = rows2[row0, col] * w2[0, gs]
                for n in range(1, 8):
                    a = a + rows2[row0 + n * SB, col] * w2[n, gs]
                outv[l, gs] = a
                return _

            lax.fori_loop(0, NG, acc_body, None)

        def fused(l):
            def fused_body(g, _, l=l):
                gs, iv, wv = corner_math(g, l)
                a = plsc.load_gather(tabv, [iv(0)]) * wv(0)
                for n in range(1, 8):
                    a = a + plsc.load_gather(tabv, [iv(n)]) * wv(n)
                outv[l, gs] = a
                return _

            lax.fori_loop(0, NG, fused_body, None)

        compute(STREAM_LEVELS[0], 0)
        fire(STREAM_LEVELS[0], 0)
        compute(STREAM_LEVELS[1], 1)
        fire(STREAM_LEVELS[1], 1)
        for l in LOC_LEVELS:
            fused(l)
        for i, l in enumerate(STREAM_LEVELS):
            s = i & 1
            drain(s)
            acc(l, s)
            if i + 2 < len(STREAM_LEVELS):
                compute(STREAM_LEVELS[i + 2], s)
                fire(STREAM_LEVELS[i + 2], s)

        dst0 = pl.multiple_of((wid * NCH + ch) * N_LEVELS, 8)
        pltpu.sync_copy(outv, out.at[pl.ds(dst0, N_LEVELS), :])
        return _

    lax.fori_loop(0, NCH, chunk_body, None)


@functools.partial(
    pl.kernel,
    mesh=plsc.VectorSubcoreMesh(core_axis_name="c", subcore_axis_name="s"),
    compiler_params=pltpu.CompilerParams(needs_layout_passes=False),
    out_type=jax.ShapeDtypeStruct((NW * NCH * N_LEVELS, C2), jnp.float32),
    scratch_types=[
        pltpu.VMEM((C2,), jnp.float32),
        pltpu.VMEM((C2,), jnp.float32),
        pltpu.VMEM((C2,), jnp.float32),
        pltpu.VMEM((LOC_WORDS,), jnp.float32),
        pltpu.VMEM_SHARED((SPM_WORDS,), jnp.float32),
        pltpu.VMEM((BS,), jnp.float32),
        pltpu.VMEM((8, C2), jnp.float32),
        pltpu.VMEM((8, C2), jnp.float32),
        pltpu.VMEM((N_LEVELS, C2), jnp.float32),
        pltpu.SemaphoreType.DMA,
        pltpu.SemaphoreType.DMA,
        pltpu.VMEM((8 * SB, 128), jnp.int32),
        pltpu.VMEM((8 * SB, 128), jnp.int32),
        pltpu.VMEM((8 * SB, 128), jnp.float32),
        pltpu.VMEM((8 * SB, 128), jnp.float32),
    ],
)
def _encode(xd, yd, zd, tflat, out, *scratch):
    _body(xd, yd, zd, tflat, out, *scratch)


def kernel(points, lattice_type, table):
    del lattice_type
    dup = jnp.repeat(points, 2, axis=0).reshape(points.shape[0], 2, 3)
    xd = dup[:, :, 0].reshape(-1)
    yd = dup[:, :, 1].reshape(-1)
    zd = dup[:, :, 2].reshape(-1)
    tflat = table.reshape(-1)
    out = _encode(xd, yd, zd, tflat)
    out = out.reshape(NW * NCH, N_LEVELS, C, 2)
    out = out.transpose(0, 2, 1, 3)
    return out.reshape(N_PTS, N_LEVELS * N_FEATS)
